# baseline (speedup 1.0000x reference)
import functools

import jax
import jax.numpy as jnp
from jax import lax
from jax.experimental import pallas as pl
from jax.experimental.pallas import tpu as pltpu
from jax.experimental.pallas import tpu_sc as plsc

EMBED = 64
HIST = 50
G = 128
NC, NS = 2, 16
NW = NC * NS


NBUF = 4


def _body(xt_hbm, w_hbm, out_hbm, xv, rows_v, in_sems, out_sems, bpw):
    batch = bpw * NW
    kph = bpw // G
    nsteps = HIST * kph
    wid = lax.axis_index("s") * NC + lax.axis_index("c")
    b0 = wid * bpw
    pltpu.sync_copy(xt_hbm.at[:, pl.ds(b0, bpw)], xv)

    zeros16 = jnp.zeros((16,), jnp.float32)

    def dbl(i, carry):
        h = i // (bpw // 16)
        kb = (i % (bpw // 16)) * 16
        v = xv[h, pl.ds(kb, 16)]
        xv[h, pl.ds(kb, 16)] = v + v
        return carry

    lax.fori_loop(0, HIST * (bpw // 16), dbl, 0)

    def issue_gather(t):
        h = t // kph
        kb = (t % kph) * G
        pltpu.async_copy(
            w_hbm.at[xv.at[h, pl.ds(kb, G)]],
            rows_v.at[t % NBUF],
            in_sems.at[t % NBUF],
        )

    def step(t, carry):
        h = t // kph
        kb = (t % kph) * G
        slot = t % NBUF
        pltpu.make_async_copy(
            w_hbm.at[xv.at[h, pl.ds(kb, G)]], rows_v.at[slot], in_sems.at[slot]
        ).wait()
        m = xv[h, pl.ds(kb, 16)]
        for q in range(1, G // 16):
            m = jnp.minimum(m, xv[h, pl.ds(kb + q * 16, 16)])
        gmin = m[0]
        for i in range(1, 16):
            gmin = jnp.minimum(gmin, m[i])

        @pl.when(gmin <= 0)
        def _fixup():
            for sg in range(G // 16):
                iv = xv[h, pl.ds(kb + sg * 16, 16)]
                for i in range(16):

                    @pl.when(iv[i] <= 0)
                    def _zero_row(sg=sg, i=i):
                        for q in range(EMBED // 16):
                            rows_v[slot, sg * 16 + i, pl.ds(q * 16, 16)] = zeros16

        dst = out_hbm.at[pl.ds(h * batch + b0 + kb, G)]
        pltpu.async_copy(rows_v.at[slot], dst, out_sems.at[slot])

        @pl.when(t + 2 < nsteps)
        def _ahead():
            @pl.when(t >= 2)
            def _drain():
                t2 = t - 2
                h2 = t2 // kph
                kb2 = (t2 % kph) * G
                dst2 = out_hbm.at[pl.ds(h2 * batch + b0 + kb2, G)]
                pltpu.make_async_copy(
                    rows_v.at[t2 % NBUF], dst2, out_sems.at[t2 % NBUF]
                ).wait()

            issue_gather(t + 2)

        return carry

    issue_gather(0)
    issue_gather(1)
    lax.fori_loop(0, nsteps, step, 0)

    for tl in range(nsteps - NBUF, nsteps):
        h2 = tl // kph
        kb2 = (tl % kph) * G
        src = out_hbm.at[pl.ds(h2 * batch + b0 + kb2, G)]
        pltpu.make_async_copy(
            rows_v.at[tl % NBUF], src, out_sems.at[tl % NBUF]
        ).wait()


_WT_BC = 1024


def _wt_body(wt_ref, o_ref):
    x = wt_ref[...]
    sel = jnp.eye(EMBED, 2 * EMBED, dtype=jnp.float32)
    o_ref[...] = lax.dot_general(x, sel, (((0,), (0,)), ((), ())),
                                 preferred_element_type=jnp.float32)


def _w_pad_row_major(wt):
    vocab = wt.shape[1]
    return pl.pallas_call(
        _wt_body,
        grid=(pl.cdiv(vocab, _WT_BC),),
        in_specs=[pl.BlockSpec((EMBED, _WT_BC), lambda j: (0, j))],
        out_specs=pl.BlockSpec((_WT_BC, 2 * EMBED), lambda j: (j, 0)),
        out_shape=jax.ShapeDtypeStruct((vocab, 2 * EMBED), jnp.float32),
    )(wt)


@functools.partial(jax.jit, static_argnames=("bpw",))
def _embed_call(xt, w, bpw):
    batch = bpw * NW
    mesh = plsc.VectorSubcoreMesh(core_axis_name="c", subcore_axis_name="s")
    return pl.kernel(
        functools.partial(_body, bpw=bpw),
        out_type=jax.ShapeDtypeStruct((HIST * batch, EMBED), jnp.float32),
        mesh=mesh,
        scratch_types=[
            pltpu.VMEM((HIST, bpw), jnp.int32),
            pltpu.VMEM((NBUF, G, EMBED), jnp.float32),
            pltpu.SemaphoreType.DMA((NBUF,)),
            pltpu.SemaphoreType.DMA((NBUF,)),
        ],
        compiler_params=pltpu.CompilerParams(use_tc_tiling_on_sc=False),
    )(xt, w)


def kernel(x, W):
    b, h, _ = x.shape
    assert h == HIST and b % (NW * G) == 0
    bpw = b // NW
    xt = jnp.transpose(x, (1, 0, 2))[:, :, 0].astype(jnp.int32)
    w2 = _w_pad_row_major(jnp.transpose(W)).reshape(2 * W.shape[0], EMBED)
    out = _embed_call(xt, w2, bpw)
    return jnp.transpose(out.reshape(HIST, b, EMBED), (1, 0, 2))


# speedup vs baseline: 1.4135x; 1.4135x over previous
import functools

import jax
import jax.numpy as jnp
from jax import lax
from jax.experimental import pallas as pl
from jax.experimental.pallas import tpu as pltpu
from jax.experimental.pallas import tpu_sc as plsc

EMBED = 64
HIST = 50
G = 128
NC, NS = 2, 16
NW = NC * NS


NBUF = 4


def _body(xt_hbm, w_hbm, out_hbm, xv, rows_v, in_sems, out_sems, bpw):
    batch = bpw * NW
    kph = bpw // G
    nsteps = HIST * kph
    wid = lax.axis_index("s") * NC + lax.axis_index("c")
    b0 = wid * bpw
    pltpu.sync_copy(xt_hbm.at[:, pl.ds(b0, bpw)], xv)

    zeros16 = jnp.zeros((16,), jnp.float32)

    def dbl(i, carry):
        h = i // (bpw // 16)
        kb = (i % (bpw // 16)) * 16
        v = xv[h, pl.ds(kb, 16)]
        xv[h, pl.ds(kb, 16)] = v + v
        return carry

    lax.fori_loop(0, HIST * (bpw // 16), dbl, 0)

    def issue_gather(t):
        h = t // kph
        kb = (t % kph) * G
        pltpu.async_copy(
            w_hbm.at[xv.at[h, pl.ds(kb, G)]],
            rows_v.at[t % NBUF],
            in_sems.at[t % NBUF],
        )

    def step(t, carry):
        h = t // kph
        kb = (t % kph) * G
        slot = t % NBUF
        pltpu.make_async_copy(
            w_hbm.at[xv.at[h, pl.ds(kb, G)]], rows_v.at[slot], in_sems.at[slot]
        ).wait()
        m = xv[h, pl.ds(kb, 16)]
        for q in range(1, G // 16):
            m = jnp.minimum(m, xv[h, pl.ds(kb + q * 16, 16)])
        gmin = m[0]
        for i in range(1, 16):
            gmin = jnp.minimum(gmin, m[i])

        @pl.when(gmin <= 0)
        def _fixup():
            for sg in range(G // 16):
                iv = xv[h, pl.ds(kb + sg * 16, 16)]
                for i in range(16):

                    @pl.when(iv[i] <= 0)
                    def _zero_row(sg=sg, i=i):
                        for q in range(EMBED // 16):
                            rows_v[slot, sg * 16 + i, pl.ds(q * 16, 16)] = zeros16

        dst = out_hbm.at[pl.ds(h * batch + b0 + kb, G)]
        pltpu.async_copy(rows_v.at[slot], dst, out_sems.at[slot])

        @pl.when(t + 2 < nsteps)
        def _ahead():
            @pl.when(t >= 2)
            def _drain():
                t2 = t - 2
                h2 = t2 // kph
                kb2 = (t2 % kph) * G
                dst2 = out_hbm.at[pl.ds(h2 * batch + b0 + kb2, G)]
                pltpu.make_async_copy(
                    rows_v.at[t2 % NBUF], dst2, out_sems.at[t2 % NBUF]
                ).wait()

            issue_gather(t + 2)

        return carry

    issue_gather(0)
    issue_gather(1)
    lax.fori_loop(0, nsteps, step, 0)

    for tl in range(nsteps - NBUF, nsteps):
        h2 = tl // kph
        kb2 = (tl % kph) * G
        src = out_hbm.at[pl.ds(h2 * batch + b0 + kb2, G)]
        pltpu.make_async_copy(
            rows_v.at[tl % NBUF], src, out_sems.at[tl % NBUF]
        ).wait()


_WT_BC = 4096


def _wt_body(wt_ref, o_ref):
    x = wt_ref[...]
    sel = jnp.eye(EMBED, 2 * EMBED, dtype=jnp.float32)
    o_ref[...] = lax.dot_general(x, sel, (((0,), (0,)), ((), ())),
                                 preferred_element_type=jnp.float32)


def _w_pad_row_major(wt):
    vocab = wt.shape[1]
    return pl.pallas_call(
        _wt_body,
        grid=(pl.cdiv(vocab, _WT_BC),),
        in_specs=[pl.BlockSpec((EMBED, _WT_BC), lambda j: (0, j))],
        out_specs=pl.BlockSpec((_WT_BC, 2 * EMBED), lambda j: (j, 0)),
        out_shape=jax.ShapeDtypeStruct((vocab, 2 * EMBED), jnp.float32),
    )(wt)


@functools.partial(jax.jit, static_argnames=("bpw",))
def _embed_call(xt, w, bpw):
    batch = bpw * NW
    mesh = plsc.VectorSubcoreMesh(core_axis_name="c", subcore_axis_name="s")
    return pl.kernel(
        functools.partial(_body, bpw=bpw),
        out_type=jax.ShapeDtypeStruct((HIST * batch, EMBED), jnp.float32),
        mesh=mesh,
        scratch_types=[
            pltpu.VMEM((HIST, bpw), jnp.int32),
            pltpu.VMEM((NBUF, G, EMBED), jnp.float32),
            pltpu.SemaphoreType.DMA((NBUF,)),
            pltpu.SemaphoreType.DMA((NBUF,)),
        ],
        compiler_params=pltpu.CompilerParams(use_tc_tiling_on_sc=False),
    )(xt, w)


def kernel(x, W):
    b, h, _ = x.shape
    assert h == HIST and b % (NW * G) == 0
    bpw = b // NW
    xt = jnp.transpose(x, (1, 0, 2))[:, :, 0].astype(jnp.int32)
    w2 = _w_pad_row_major(jnp.transpose(W)).reshape(2 * W.shape[0], EMBED)
    out = _embed_call(xt, w2, bpw)
    return jnp.transpose(out.reshape(HIST, b, EMBED), (1, 0, 2))


# speedup vs baseline: 1.5328x; 1.0844x over previous
import functools

import jax
import jax.numpy as jnp
from jax import lax
from jax.experimental import pallas as pl
from jax.experimental.pallas import tpu as pltpu
from jax.experimental.pallas import tpu_sc as plsc

EMBED = 64
HIST = 50
G = 128
NC, NS = 2, 16
NW = NC * NS


NBUF = 4


def _body(xt_hbm, w_hbm, out_hbm, xv, rows_v, in_sems, out_sems, bpw):
    batch = bpw * NW
    kph = bpw // G
    nsteps = HIST * kph
    wid = lax.axis_index("s") * NC + lax.axis_index("c")
    b0 = wid * bpw
    pltpu.sync_copy(xt_hbm.at[:, pl.ds(b0, bpw)], xv)

    zeros16 = jnp.zeros((16,), jnp.float32)

    def dbl(i, carry):
        h = i // (bpw // 16)
        kb = (i % (bpw // 16)) * 16
        v = xv[h, pl.ds(kb, 16)]
        xv[h, pl.ds(kb, 16)] = v + v
        return carry

    lax.fori_loop(0, HIST * (bpw // 16), dbl, 0)

    def issue_gather(t):
        h = t // kph
        kb = (t % kph) * G
        pltpu.async_copy(
            w_hbm.at[xv.at[h, pl.ds(kb, G)]],
            rows_v.at[t % NBUF],
            in_sems.at[t % NBUF],
        )

    def step(t, carry):
        h = t // kph
        kb = (t % kph) * G
        slot = t % NBUF
        pltpu.make_async_copy(
            w_hbm.at[xv.at[h, pl.ds(kb, G)]], rows_v.at[slot], in_sems.at[slot]
        ).wait()
        m = xv[h, pl.ds(kb, 16)]
        for q in range(1, G // 16):
            m = jnp.minimum(m, xv[h, pl.ds(kb + q * 16, 16)])
        gmin = m[0]
        for i in range(1, 16):
            gmin = jnp.minimum(gmin, m[i])

        @pl.when(gmin <= 0)
        def _fixup():
            for sg in range(G // 16):
                iv = xv[h, pl.ds(kb + sg * 16, 16)]
                for i in range(16):

                    @pl.when(iv[i] <= 0)
                    def _zero_row(sg=sg, i=i):
                        for q in range(EMBED // 16):
                            rows_v[slot, sg * 16 + i, pl.ds(q * 16, 16)] = zeros16

        dst = out_hbm.at[pl.ds(h * batch + b0 + kb, G)]
        pltpu.async_copy(rows_v.at[slot], dst, out_sems.at[slot])

        @pl.when(t + 2 < nsteps)
        def _ahead():
            @pl.when(t >= 2)
            def _drain():
                t2 = t - 2
                h2 = t2 // kph
                kb2 = (t2 % kph) * G
                dst2 = out_hbm.at[pl.ds(h2 * batch + b0 + kb2, G)]
                pltpu.make_async_copy(
                    rows_v.at[t2 % NBUF], dst2, out_sems.at[t2 % NBUF]
                ).wait()

            issue_gather(t + 2)

        return carry

    issue_gather(0)
    issue_gather(1)
    lax.fori_loop(0, nsteps, step, 0)

    for tl in range(nsteps - NBUF, nsteps):
        h2 = tl // kph
        kb2 = (tl % kph) * G
        src = out_hbm.at[pl.ds(h2 * batch + b0 + kb2, G)]
        pltpu.make_async_copy(
            rows_v.at[tl % NBUF], src, out_sems.at[tl % NBUF]
        ).wait()


_WT_BC = 8192


def _wt_body(wt_ref, o_ref):
    x = wt_ref[...]
    sel = jnp.eye(EMBED, 2 * EMBED, dtype=jnp.float32)
    o_ref[...] = lax.dot_general(x, sel, (((0,), (0,)), ((), ())),
                                 preferred_element_type=jnp.float32)


def _w_pad_row_major(wt):
    vocab = wt.shape[1]
    return pl.pallas_call(
        _wt_body,
        grid=(pl.cdiv(vocab, _WT_BC),),
        in_specs=[pl.BlockSpec((EMBED, _WT_BC), lambda j: (0, j))],
        out_specs=pl.BlockSpec((_WT_BC, 2 * EMBED), lambda j: (j, 0)),
        out_shape=jax.ShapeDtypeStruct((vocab, 2 * EMBED), jnp.float32),
    )(wt)


@functools.partial(jax.jit, static_argnames=("bpw",))
def _embed_call(xt, w, bpw):
    batch = bpw * NW
    mesh = plsc.VectorSubcoreMesh(core_axis_name="c", subcore_axis_name="s")
    return pl.kernel(
        functools.partial(_body, bpw=bpw),
        out_type=jax.ShapeDtypeStruct((HIST * batch, EMBED), jnp.float32),
        mesh=mesh,
        scratch_types=[
            pltpu.VMEM((HIST, bpw), jnp.int32),
            pltpu.VMEM((NBUF, G, EMBED), jnp.float32),
            pltpu.SemaphoreType.DMA((NBUF,)),
            pltpu.SemaphoreType.DMA((NBUF,)),
        ],
        compiler_params=pltpu.CompilerParams(use_tc_tiling_on_sc=False),
    )(xt, w)


def kernel(x, W):
    b, h, _ = x.shape
    assert h == HIST and b % (NW * G) == 0
    bpw = b // NW
    xt = jnp.transpose(x, (1, 0, 2))[:, :, 0].astype(jnp.int32)
    w2 = _w_pad_row_major(jnp.transpose(W)).reshape(2 * W.shape[0], EMBED)
    out = _embed_call(xt, w2, bpw)
    return jnp.transpose(out.reshape(HIST, b, EMBED), (1, 0, 2))


# speedup vs baseline: 1.5534x; 1.0135x over previous
import functools

import jax
import jax.numpy as jnp
from jax import lax
from jax.experimental import pallas as pl
from jax.experimental.pallas import tpu as pltpu
from jax.experimental.pallas import tpu_sc as plsc

EMBED = 64
HIST = 50
G = 128
NC, NS = 2, 16
NW = NC * NS


NBUF = 6
LEAD = 3


def _body(xt_hbm, w_hbm, out_hbm, xv, rows_v, in_sems, out_sems, bpw):
    batch = bpw * NW
    kph = bpw // G
    nsteps = HIST * kph
    wid = lax.axis_index("s") * NC + lax.axis_index("c")
    b0 = wid * bpw
    pltpu.sync_copy(xt_hbm.at[:, pl.ds(b0, bpw)], xv)

    zeros16 = jnp.zeros((16,), jnp.float32)

    def dbl(i, carry):
        h = i // (bpw // 16)
        kb = (i % (bpw // 16)) * 16
        v = xv[h, pl.ds(kb, 16)]
        xv[h, pl.ds(kb, 16)] = v + v
        return carry

    lax.fori_loop(0, HIST * (bpw // 16), dbl, 0)

    def issue_gather(t):
        h = t // kph
        kb = (t % kph) * G
        pltpu.async_copy(
            w_hbm.at[xv.at[h, pl.ds(kb, G)]],
            rows_v.at[t % NBUF],
            in_sems.at[t % NBUF],
        )

    def step(t, carry):
        h = t // kph
        kb = (t % kph) * G
        slot = t % NBUF
        pltpu.make_async_copy(
            w_hbm.at[xv.at[h, pl.ds(kb, G)]], rows_v.at[slot], in_sems.at[slot]
        ).wait()
        m = xv[h, pl.ds(kb, 16)]
        for q in range(1, G // 16):
            m = jnp.minimum(m, xv[h, pl.ds(kb + q * 16, 16)])
        gmin = m[0]
        for i in range(1, 16):
            gmin = jnp.minimum(gmin, m[i])

        @pl.when(gmin <= 0)
        def _fixup():
            for sg in range(G // 16):
                iv = xv[h, pl.ds(kb + sg * 16, 16)]
                for i in range(16):

                    @pl.when(iv[i] <= 0)
                    def _zero_row(sg=sg, i=i):
                        for q in range(EMBED // 16):
                            rows_v[slot, sg * 16 + i, pl.ds(q * 16, 16)] = zeros16

        dst = out_hbm.at[pl.ds(h * batch + b0 + kb, G)]
        pltpu.async_copy(rows_v.at[slot], dst, out_sems.at[slot])

        @pl.when(t + LEAD < nsteps)
        def _ahead():
            @pl.when(t + LEAD >= NBUF)
            def _drain():
                t2 = t + LEAD - NBUF
                h2 = t2 // kph
                kb2 = (t2 % kph) * G
                dst2 = out_hbm.at[pl.ds(h2 * batch + b0 + kb2, G)]
                pltpu.make_async_copy(
                    rows_v.at[t2 % NBUF], dst2, out_sems.at[t2 % NBUF]
                ).wait()

            issue_gather(t + LEAD)

        return carry

    for tp in range(LEAD):
        issue_gather(tp)
    lax.fori_loop(0, nsteps, step, 0)

    for tl in range(nsteps - NBUF, nsteps):
        h2 = tl // kph
        kb2 = (tl % kph) * G
        src = out_hbm.at[pl.ds(h2 * batch + b0 + kb2, G)]
        pltpu.make_async_copy(
            rows_v.at[tl % NBUF], src, out_sems.at[tl % NBUF]
        ).wait()


_WT_BC = 8192


def _wt_body(wt_ref, o_ref):
    x = wt_ref[...]
    sel = jnp.eye(EMBED, 2 * EMBED, dtype=jnp.float32)
    o_ref[...] = lax.dot_general(x, sel, (((0,), (0,)), ((), ())),
                                 preferred_element_type=jnp.float32)


def _w_pad_row_major(wt):
    vocab = wt.shape[1]
    return pl.pallas_call(
        _wt_body,
        grid=(pl.cdiv(vocab, _WT_BC),),
        in_specs=[pl.BlockSpec((EMBED, _WT_BC), lambda j: (0, j))],
        out_specs=pl.BlockSpec((_WT_BC, 2 * EMBED), lambda j: (j, 0)),
        out_shape=jax.ShapeDtypeStruct((vocab, 2 * EMBED), jnp.float32),
    )(wt)


@functools.partial(jax.jit, static_argnames=("bpw",))
def _embed_call(xt, w, bpw):
    batch = bpw * NW
    mesh = plsc.VectorSubcoreMesh(core_axis_name="c", subcore_axis_name="s")
    return pl.kernel(
        functools.partial(_body, bpw=bpw),
        out_type=jax.ShapeDtypeStruct((HIST * batch, EMBED), jnp.float32),
        mesh=mesh,
        scratch_types=[
            pltpu.VMEM((HIST, bpw), jnp.int32),
            pltpu.VMEM((NBUF, G, EMBED), jnp.float32),
            pltpu.SemaphoreType.DMA((NBUF,)),
            pltpu.SemaphoreType.DMA((NBUF,)),
        ],
        compiler_params=pltpu.CompilerParams(use_tc_tiling_on_sc=False),
    )(xt, w)


def kernel(x, W):
    b, h, _ = x.shape
    assert h == HIST and b % (NW * G) == 0
    bpw = b // NW
    xt = jnp.transpose(x, (1, 0, 2))[:, :, 0].astype(jnp.int32)
    w2 = _w_pad_row_major(jnp.transpose(W)).reshape(2 * W.shape[0], EMBED)
    out = _embed_call(xt, w2, bpw)
    return jnp.transpose(out.reshape(HIST, b, EMBED), (1, 0, 2))


# speedup vs baseline: 1.5979x; 1.0286x over previous
import functools

import jax
import jax.numpy as jnp
from jax import lax
from jax.experimental import pallas as pl
from jax.experimental.pallas import tpu as pltpu
from jax.experimental.pallas import tpu_sc as plsc

EMBED = 64
HIST = 50
G = 128
NC, NS = 2, 16
NW = NC * NS


NBUF = 6
LEAD = 3


def _body(xt_hbm, w_hbm, out_hbm, xv, rows_v, in_sems, out_sems, bpw):
    batch = bpw * NW
    kph = bpw // G
    nsteps = HIST * kph
    wid = lax.axis_index("s") * NC + lax.axis_index("c")
    b0 = wid * bpw
    pltpu.sync_copy(xt_hbm.at[:, pl.ds(b0, bpw)], xv)

    zeros16 = jnp.zeros((16,), jnp.float32)

    def dbl(i, carry):
        h = i // (bpw // 16)
        kb = (i % (bpw // 16)) * 16
        v = xv[h, pl.ds(kb, 16)]
        xv[h, pl.ds(kb, 16)] = v + v
        return carry

    lax.fori_loop(0, HIST * (bpw // 16), dbl, 0)

    def issue_gather(t):
        h = t // kph
        kb = (t % kph) * G
        pltpu.async_copy(
            w_hbm.at[xv.at[h, pl.ds(kb, G)]],
            rows_v.at[t % NBUF],
            in_sems.at[t % NBUF],
        )

    def step(t, carry):
        h = t // kph
        kb = (t % kph) * G
        slot = t % NBUF
        pltpu.make_async_copy(
            w_hbm.at[xv.at[h, pl.ds(kb, G)]], rows_v.at[slot], in_sems.at[slot]
        ).wait()
        m = xv[h, pl.ds(kb, 16)]
        for q in range(1, G // 16):
            m = jnp.minimum(m, xv[h, pl.ds(kb + q * 16, 16)])
        gmin = m[0]
        for i in range(1, 16):
            gmin = jnp.minimum(gmin, m[i])

        @pl.when(gmin <= 0)
        def _fixup():
            for sg in range(G // 16):
                iv = xv[h, pl.ds(kb + sg * 16, 16)]
                for i in range(16):

                    @pl.when(iv[i] <= 0)
                    def _zero_row(sg=sg, i=i):
                        for q in range(EMBED // 16):
                            rows_v[slot, sg * 16 + i, pl.ds(q * 16, 16)] = zeros16

        dst = out_hbm.at[pl.ds(h * batch + b0 + kb, G)]
        pltpu.async_copy(rows_v.at[slot], dst, out_sems.at[slot])

        @pl.when(t + LEAD < nsteps)
        def _ahead():
            @pl.when(t + LEAD >= NBUF)
            def _drain():
                t2 = t + LEAD - NBUF
                h2 = t2 // kph
                kb2 = (t2 % kph) * G
                dst2 = out_hbm.at[pl.ds(h2 * batch + b0 + kb2, G)]
                pltpu.make_async_copy(
                    rows_v.at[t2 % NBUF], dst2, out_sems.at[t2 % NBUF]
                ).wait()

            issue_gather(t + LEAD)

        return carry

    for tp in range(LEAD):
        issue_gather(tp)
    lax.fori_loop(0, nsteps, step, 0)

    for tl in range(nsteps - NBUF, nsteps):
        h2 = tl // kph
        kb2 = (tl % kph) * G
        src = out_hbm.at[pl.ds(h2 * batch + b0 + kb2, G)]
        pltpu.make_async_copy(
            rows_v.at[tl % NBUF], src, out_sems.at[tl % NBUF]
        ).wait()


_WT_BC = 16384


def _wt_body(wt_ref, o_ref):
    x = wt_ref[...]
    sel = jnp.eye(EMBED, 2 * EMBED, dtype=jnp.float32)
    o_ref[...] = lax.dot_general(x, sel, (((0,), (0,)), ((), ())),
                                 preferred_element_type=jnp.float32)


def _w_pad_row_major(wt):
    vocab = wt.shape[1]
    return pl.pallas_call(
        _wt_body,
        grid=(pl.cdiv(vocab, _WT_BC),),
        in_specs=[pl.BlockSpec((EMBED, _WT_BC), lambda j: (0, j))],
        out_specs=pl.BlockSpec((_WT_BC, 2 * EMBED), lambda j: (j, 0)),
        out_shape=jax.ShapeDtypeStruct((vocab, 2 * EMBED), jnp.float32),
    )(wt)


@functools.partial(jax.jit, static_argnames=("bpw",))
def _embed_call(xt, w, bpw):
    batch = bpw * NW
    mesh = plsc.VectorSubcoreMesh(core_axis_name="c", subcore_axis_name="s")
    return pl.kernel(
        functools.partial(_body, bpw=bpw),
        out_type=jax.ShapeDtypeStruct((HIST * batch, EMBED), jnp.float32),
        mesh=mesh,
        scratch_types=[
            pltpu.VMEM((HIST, bpw), jnp.int32),
            pltpu.VMEM((NBUF, G, EMBED), jnp.float32),
            pltpu.SemaphoreType.DMA((NBUF,)),
            pltpu.SemaphoreType.DMA((NBUF,)),
        ],
        compiler_params=pltpu.CompilerParams(use_tc_tiling_on_sc=False),
    )(xt, w)


def kernel(x, W):
    b, h, _ = x.shape
    assert h == HIST and b % (NW * G) == 0
    bpw = b // NW
    xt = jnp.transpose(x, (1, 0, 2))[:, :, 0].astype(jnp.int32)
    w2 = _w_pad_row_major(jnp.transpose(W)).reshape(2 * W.shape[0], EMBED)
    out = _embed_call(xt, w2, bpw)
    return jnp.transpose(out.reshape(HIST, b, EMBED), (1, 0, 2))


# speedup vs baseline: 1.6084x; 1.0066x over previous
import functools

import jax
import jax.numpy as jnp
from jax import lax
from jax.experimental import pallas as pl
from jax.experimental.pallas import tpu as pltpu
from jax.experimental.pallas import tpu_sc as plsc

EMBED = 64
HIST = 50
G = 128
NC, NS = 2, 16
NW = NC * NS


NBUF = 6
LEAD = 3


def _body(xt_hbm, w_hbm, out_hbm, xv, rows_v, in_sems, out_sems, bpw):
    batch = bpw * NW
    kph = bpw // G
    nsteps = HIST * kph
    wid = lax.axis_index("s") * NC + lax.axis_index("c")
    b0 = wid * bpw
    pltpu.sync_copy(xt_hbm.at[:, pl.ds(b0, bpw)], xv)

    zeros16 = jnp.zeros((16,), jnp.float32)

    def dbl(i, carry):
        h = i // (bpw // 16)
        kb = (i % (bpw // 16)) * 16
        v = xv[h, pl.ds(kb, 16)]
        xv[h, pl.ds(kb, 16)] = v + v
        return carry

    lax.fori_loop(0, HIST * (bpw // 16), dbl, 0)

    def issue_gather(t):
        h = t // kph
        kb = (t % kph) * G
        pltpu.async_copy(
            w_hbm.at[xv.at[h, pl.ds(kb, G)]],
            rows_v.at[t % NBUF],
            in_sems.at[t % NBUF],
        )

    def step(t, carry):
        h = t // kph
        kb = (t % kph) * G
        slot = t % NBUF
        pltpu.make_async_copy(
            w_hbm.at[xv.at[h, pl.ds(kb, G)]], rows_v.at[slot], in_sems.at[slot]
        ).wait()
        m = xv[h, pl.ds(kb, 16)]
        for q in range(1, G // 16):
            m = jnp.minimum(m, xv[h, pl.ds(kb + q * 16, 16)])
        gmin = m[0]
        for i in range(1, 16):
            gmin = jnp.minimum(gmin, m[i])

        @pl.when(gmin <= 0)
        def _fixup():
            for sg in range(G // 16):
                iv = xv[h, pl.ds(kb + sg * 16, 16)]
                for i in range(16):

                    @pl.when(iv[i] <= 0)
                    def _zero_row(sg=sg, i=i):
                        for q in range(EMBED // 16):
                            rows_v[slot, sg * 16 + i, pl.ds(q * 16, 16)] = zeros16

        dst = out_hbm.at[pl.ds(h * batch + b0 + kb, G)]
        pltpu.async_copy(rows_v.at[slot], dst, out_sems.at[slot])

        @pl.when(t + LEAD < nsteps)
        def _ahead():
            @pl.when(t + LEAD >= NBUF)
            def _drain():
                t2 = t + LEAD - NBUF
                h2 = t2 // kph
                kb2 = (t2 % kph) * G
                dst2 = out_hbm.at[pl.ds(h2 * batch + b0 + kb2, G)]
                pltpu.make_async_copy(
                    rows_v.at[t2 % NBUF], dst2, out_sems.at[t2 % NBUF]
                ).wait()

            issue_gather(t + LEAD)

        return carry

    for tp in range(LEAD):
        issue_gather(tp)
    lax.fori_loop(0, nsteps, step, 0)

    for tl in range(nsteps - NBUF, nsteps):
        h2 = tl // kph
        kb2 = (tl % kph) * G
        src = out_hbm.at[pl.ds(h2 * batch + b0 + kb2, G)]
        pltpu.make_async_copy(
            rows_v.at[tl % NBUF], src, out_sems.at[tl % NBUF]
        ).wait()


_WT_BC = 32768


def _wt_body(wt_ref, o_ref):
    x = wt_ref[...]
    sel = jnp.eye(EMBED, 2 * EMBED, dtype=jnp.float32)
    o_ref[...] = lax.dot_general(x, sel, (((0,), (0,)), ((), ())),
                                 preferred_element_type=jnp.float32)


def _w_pad_row_major(wt):
    vocab = wt.shape[1]
    return pl.pallas_call(
        _wt_body,
        grid=(pl.cdiv(vocab, _WT_BC),),
        in_specs=[pl.BlockSpec((EMBED, _WT_BC), lambda j: (0, j))],
        out_specs=pl.BlockSpec((_WT_BC, 2 * EMBED), lambda j: (j, 0)),
        out_shape=jax.ShapeDtypeStruct((vocab, 2 * EMBED), jnp.float32),
    )(wt)


@functools.partial(jax.jit, static_argnames=("bpw",))
def _embed_call(xt, w, bpw):
    batch = bpw * NW
    mesh = plsc.VectorSubcoreMesh(core_axis_name="c", subcore_axis_name="s")
    return pl.kernel(
        functools.partial(_body, bpw=bpw),
        out_type=jax.ShapeDtypeStruct((HIST * batch, EMBED), jnp.float32),
        mesh=mesh,
        scratch_types=[
            pltpu.VMEM((HIST, bpw), jnp.int32),
            pltpu.VMEM((NBUF, G, EMBED), jnp.float32),
            pltpu.SemaphoreType.DMA((NBUF,)),
            pltpu.SemaphoreType.DMA((NBUF,)),
        ],
        compiler_params=pltpu.CompilerParams(use_tc_tiling_on_sc=False),
    )(xt, w)


def kernel(x, W):
    b, h, _ = x.shape
    assert h == HIST and b % (NW * G) == 0
    bpw = b // NW
    xt = jnp.transpose(x, (1, 0, 2))[:, :, 0].astype(jnp.int32)
    w2 = _w_pad_row_major(jnp.transpose(W)).reshape(2 * W.shape[0], EMBED)
    out = _embed_call(xt, w2, bpw)
    return jnp.transpose(out.reshape(HIST, b, EMBED), (1, 0, 2))
